# baseline (speedup 1.0000x reference)
import jax
import jax.numpy as jnp
from jax.experimental import pallas as pl

BLK = 128


def kernel(x, i, t, pad_lengths, decoder_embedding_weight):
    s, b, c = x.shape
    dt, _ = decoder_embedding_weight.shape
    dims = i.shape[2]
    bd = b * dims
    n_dt = dt // BLK
    n_s = s // BLK
    n_total = n_dt + n_s

    t2 = t.reshape(1, b)
    pad2 = pad_lengths.reshape(1, b)
    i2 = i.reshape(s, bd)

    def body(w_ref, x_ref, i_ref, t_ref, pad_ref, xo_ref, io_ref, po_ref):
        g = pl.program_id(0)
        r = g % 3

        @pl.when(r == 2)
        def _():
            xo_ref[...] = jnp.broadcast_to(w_ref[...][:, None, :], (BLK, b, c))
            lane = jax.lax.broadcasted_iota(jnp.int32, (1, bd), 1)
            tv = jnp.repeat(t_ref[...], dims, axis=1)
            row = jnp.where(lane % dims == 0, 1,
                            jnp.where(lane % dims == 1, tv, -1))
            io_ref[...] = jnp.broadcast_to(row, (BLK, bd))

        @pl.when(r != 2)
        def _():
            xo_ref[...] = x_ref[...]
            io_ref[...] = i_ref[...]

        po_ref[...] = pad_ref[...] + dt

    def out_idx(g):
        q, r = g // 3, g % 3
        return jnp.where(r == 2, q, n_dt + 2 * q + r)

    def copy_idx(g):
        q, r = g // 3, g % 3
        return 2 * q + jnp.minimum(r, 1)

    grid = (n_total,)
    in_specs = [
        pl.BlockSpec((BLK, c), lambda g: (g // 3, 0)),
        pl.BlockSpec((BLK, b, c), lambda g: (copy_idx(g), 0, 0)),
        pl.BlockSpec((BLK, bd), lambda g: (copy_idx(g), 0)),
        pl.BlockSpec((1, b), lambda g: (0, 0)),
        pl.BlockSpec((1, b), lambda g: (0, 0)),
    ]
    out_specs = [
        pl.BlockSpec((BLK, b, c), lambda g: (out_idx(g), 0, 0)),
        pl.BlockSpec((BLK, bd), lambda g: (out_idx(g), 0)),
        pl.BlockSpec((1, b), lambda g: (0, 0)),
    ]
    out_shape = [
        jax.ShapeDtypeStruct((dt + s, b, c), x.dtype),
        jax.ShapeDtypeStruct((dt + s, bd), i.dtype),
        jax.ShapeDtypeStruct((1, b), pad_lengths.dtype),
    ]
    xo, io, po = pl.pallas_call(
        body, grid=grid, in_specs=in_specs, out_specs=out_specs,
        out_shape=out_shape,
    )(decoder_embedding_weight, x, i2, t2, pad2)
    return xo, io.reshape(dt + s, b, dims), po.reshape(b)


# speedup vs baseline: 1.0917x; 1.0917x over previous
import jax
import jax.numpy as jnp
from jax.experimental import pallas as pl
from jax.experimental.pallas import tpu as pltpu

CROWS = 256
NB = 3
BROWS = 256


def kernel(x, i, t, pad_lengths, decoder_embedding_weight):
    s, b, c = x.shape
    dt, _ = decoder_embedding_weight.shape
    dims = i.shape[2]
    bd = b * dims
    nck = s // CROWS
    nbk = dt // BROWS

    t2 = t.reshape(1, b)
    pad2 = pad_lengths.reshape(1, b)
    i2 = i.reshape(s, bd)

    def body(w_hbm, x_hbm, i_hbm, t_ref, pad_ref, xo_hbm, io_hbm, po_ref,
             w_v, cbuf, bbuf, ivm, divm,
             sem_in, sem_out, sem_b, sem_w, sem_ii, sem_io, sem_di):
        def cin(k):
            return pltpu.make_async_copy(
                x_hbm.at[pl.ds(k * CROWS, CROWS)], cbuf.at[k % NB],
                sem_in.at[k])

        def cout(k):
            return pltpu.make_async_copy(
                cbuf.at[k % NB], xo_hbm.at[pl.ds(dt + k * CROWS, CROWS)],
                sem_out.at[k])

        def bout(p):
            return pltpu.make_async_copy(
                bbuf.at[p % 2], xo_hbm.at[pl.ds(p * BROWS, BROWS)],
                sem_b.at[p])

        for k in range(NB):
            cin(k).start()
        pltpu.make_async_copy(w_hbm, w_v, sem_w).start()
        pltpu.make_async_copy(i_hbm, ivm, sem_ii).start()

        lane = jax.lax.broadcasted_iota(jnp.int32, (1, bd), 1)
        tv = jnp.repeat(t_ref[...], dims, axis=1)
        row = jnp.where(lane % dims == 0, 1,
                        jnp.where(lane % dims == 1, tv, -1))
        divm[...] = jnp.broadcast_to(row, (dt, bd))
        pltpu.make_async_copy(divm, io_hbm.at[pl.ds(0, dt)], sem_di).start()
        po_ref[...] = pad_ref[...] + dt

        pltpu.make_async_copy(i_hbm, ivm, sem_ii).wait()
        pltpu.make_async_copy(ivm, io_hbm.at[pl.ds(dt, s)], sem_io).start()
        pltpu.make_async_copy(w_hbm, w_v, sem_w).wait()

        for k in range(nck):
            cin(k).wait()
            cout(k).start()
            if k % 2 == 1:
                p = k // 2
                if p >= 2:
                    bout(p - 2).wait()
                buf = bbuf.at[p % 2]
                buf[...] = jnp.broadcast_to(
                    w_v[pl.ds(p * BROWS, BROWS), :][:, None, :],
                    (BROWS, b, c))
                bout(p).start()
            if k + NB < nck:
                cout(k).wait()
                cin(k + NB).start()

        for k in range(max(nck - NB, 0), nck):
            cout(k).wait()
        for p in range(max(nbk - 2, 0), nbk):
            bout(p).wait()
        pltpu.make_async_copy(divm, io_hbm.at[pl.ds(0, dt)], sem_di).wait()
        pltpu.make_async_copy(ivm, io_hbm.at[pl.ds(dt, s)], sem_io).wait()

    vmem = pltpu.MemorySpace.VMEM
    xo, io, po = pl.pallas_call(
        body,
        in_specs=[
            pl.BlockSpec(memory_space=pl.ANY),
            pl.BlockSpec(memory_space=pl.ANY),
            pl.BlockSpec(memory_space=pl.ANY),
            pl.BlockSpec(memory_space=vmem),
            pl.BlockSpec(memory_space=vmem),
        ],
        out_specs=[
            pl.BlockSpec(memory_space=pl.ANY),
            pl.BlockSpec(memory_space=pl.ANY),
            pl.BlockSpec(memory_space=vmem),
        ],
        out_shape=[
            jax.ShapeDtypeStruct((dt + s, b, c), x.dtype),
            jax.ShapeDtypeStruct((dt + s, bd), i.dtype),
            jax.ShapeDtypeStruct((1, b), pad_lengths.dtype),
        ],
        scratch_shapes=[
            pltpu.VMEM((dt, c), x.dtype),
            pltpu.VMEM((NB, CROWS, b, c), x.dtype),
            pltpu.VMEM((2, BROWS, b, c), x.dtype),
            pltpu.VMEM((s, bd), i.dtype),
            pltpu.VMEM((dt, bd), i.dtype),
            pltpu.SemaphoreType.DMA((s // CROWS,)),
            pltpu.SemaphoreType.DMA((s // CROWS,)),
            pltpu.SemaphoreType.DMA((dt // BROWS,)),
            pltpu.SemaphoreType.DMA,
            pltpu.SemaphoreType.DMA,
            pltpu.SemaphoreType.DMA,
            pltpu.SemaphoreType.DMA,
        ],
    )(decoder_embedding_weight, x, i2, t2, pad2)
    return xo, io.reshape(dt + s, b, dims), po.reshape(b)


# speedup vs baseline: 1.0923x; 1.0005x over previous
import jax
import jax.numpy as jnp
from jax.experimental import pallas as pl

BLK = 256


def kernel(x, i, t, pad_lengths, decoder_embedding_weight):
    s, b, c = x.shape
    dt, _ = decoder_embedding_weight.shape
    dims = i.shape[2]
    bd = b * dims
    n_dt = dt // BLK
    n_s = s // BLK
    n_total = n_dt + n_s

    t2 = t.reshape(1, b)
    pad2 = pad_lengths.reshape(1, b)
    i2 = i.reshape(s, bd)

    def body(w_ref, x_ref, i_ref, t_ref, pad_ref, xo_ref, io_ref, po_ref):
        g = pl.program_id(0)
        r = g % 3

        @pl.when(r == 2)
        def _():
            xo_ref[...] = jnp.broadcast_to(w_ref[...][:, None, :], (BLK, b, c))
            lane = jax.lax.broadcasted_iota(jnp.int32, (1, bd), 1)
            tv = jnp.repeat(t_ref[...], dims, axis=1)
            row = jnp.where(lane % dims == 0, 1,
                            jnp.where(lane % dims == 1, tv, -1))
            io_ref[...] = jnp.broadcast_to(row, (BLK, bd))

        @pl.when(r != 2)
        def _():
            xo_ref[...] = x_ref[...]
            io_ref[...] = i_ref[...]

        po_ref[...] = pad_ref[...] + dt

    def out_idx(g):
        q, r = g // 3, g % 3
        return jnp.where(r == 2, q, n_dt + 2 * q + r)

    def copy_idx(g):
        q, r = g // 3, g % 3
        return 2 * q + jnp.minimum(r, 1)

    grid = (n_total,)
    in_specs = [
        pl.BlockSpec((BLK, c), lambda g: (g // 3, 0)),
        pl.BlockSpec((BLK, b, c), lambda g: (copy_idx(g), 0, 0)),
        pl.BlockSpec((BLK, bd), lambda g: (copy_idx(g), 0)),
        pl.BlockSpec((1, b), lambda g: (0, 0)),
        pl.BlockSpec((1, b), lambda g: (0, 0)),
    ]
    out_specs = [
        pl.BlockSpec((BLK, b, c), lambda g: (out_idx(g), 0, 0)),
        pl.BlockSpec((BLK, bd), lambda g: (out_idx(g), 0)),
        pl.BlockSpec((1, b), lambda g: (0, 0)),
    ]
    out_shape = [
        jax.ShapeDtypeStruct((dt + s, b, c), x.dtype),
        jax.ShapeDtypeStruct((dt + s, bd), i.dtype),
        jax.ShapeDtypeStruct((1, b), pad_lengths.dtype),
    ]
    xo, io, po = pl.pallas_call(
        body, grid=grid, in_specs=in_specs, out_specs=out_specs,
        out_shape=out_shape,
    )(decoder_embedding_weight, x, i2, t2, pad2)
    return xo, io.reshape(dt + s, b, dims), po.reshape(b)
